# baseline (speedup 1.0000x reference)
import functools

import jax
import jax.numpy as jnp
from jax import lax
from jax.experimental import pallas as pl
from jax.experimental.pallas import tpu as pltpu, tpu_sc as plsc

N, D, M, V = 32768, 256, 8, 8192
NC, NS, L = 2, 16, 16
NW = NC * NS
ROWS_PER_W = N // NW
R = 16
BLOCKS = ROWS_PER_W // R
GIDX = 128
NG = R * M // GIDX
W = D // 2


def _sc_body(x_hbm, c_hbm, b_hbm, out_hbm,
             idx_all, rows0, rows1, x0, x1, out_v, gs0, gs1, xs0, xs1):
    wid = lax.axis_index("c") * NS + lax.axis_index("s")
    row0 = wid * ROWS_PER_W

    pltpu.sync_copy(b_hbm.at[pl.ds(row0 * M, ROWS_PER_W * M)], idx_all)

    rows = (rows0, rows1)
    xbuf = (x0, x1)
    gsem = (gs0, gs1)
    xsem = (xs0, xs1)

    def idx_slice(blk, i):
        off = pl.multiple_of(blk * (R * M) + i * GIDX, GIDX)
        return idx_all.at[pl.ds(off, GIDX)]

    def start(blk, buf):
        base = row0 + blk * R
        for i in range(NG):
            pltpu.async_copy(c_hbm.at[idx_slice(blk, i)],
                             rows[buf].at[pl.ds(i * GIDX, GIDX)], gsem[buf])
        pltpu.async_copy(x_hbm.at[pl.ds(base, R)], xbuf[buf], xsem[buf])

    def wait(blk, buf):
        for i in range(NG):
            pltpu.make_async_copy(
                c_hbm.at[idx_slice(blk, i)],
                rows[buf].at[pl.ds(i * GIDX, GIDX)], gsem[buf]).wait()
        pltpu.make_async_copy(
            x_hbm.at[pl.ds(row0, R)], xbuf[buf], xsem[buf]).wait()

    lane = lax.iota(jnp.int32, L)

    def compute(blk, buf):
        rows_v = rows[buf]
        x_v = xbuf[buf]

        def _row(r, acc):
            g = r * M
            err = jnp.zeros((L,), jnp.float32)
            for k in range(W // L):
                col = pl.ds(k * L, L)
                s = plsc.bitcast(rows_v[g, col], jnp.bfloat16)
                for j in range(1, M):
                    s = s + plsc.bitcast(rows_v[g + j, col], jnp.bfloat16)
                u = plsc.bitcast(s, jnp.int32)
                sa = plsc.bitcast(u << 16, jnp.float32)
                sb = plsc.bitcast(u & jnp.int32(-65536), jnp.float32)
                da = x_v[r, pl.ds(k * L, L)] - sa
                db = x_v[r, pl.ds(W + k * L, L)] - sb
                err = err + da * da
                err = err + db * db
            return jnp.where(lane == r, jnp.sum(err), acc)

        acc = lax.fori_loop(0, R, _row, jnp.zeros((L,), jnp.float32))
        out_v[pl.ds(blk * R, R)] = acc

    start(0, 0)
    start(1, 1)

    def _pair(p, carry):
        b0 = 2 * p
        wait(b0, 0)
        compute(b0, 0)

        @pl.when(b0 + 2 < BLOCKS)
        def _():
            start(b0 + 2, 0)

        wait(b0 + 1, 1)
        compute(b0 + 1, 1)

        @pl.when(b0 + 3 < BLOCKS)
        def _():
            start(b0 + 3, 1)

        return carry

    lax.fori_loop(0, BLOCKS // 2, _pair, 0)
    pltpu.sync_copy(out_v, out_hbm.at[pl.ds(row0, ROWS_PER_W)])


@functools.lru_cache(maxsize=1)
def _build():
    return pl.kernel(
        _sc_body,
        out_type=jax.ShapeDtypeStruct((N,), jnp.float32),
        mesh=plsc.VectorSubcoreMesh(core_axis_name="c", subcore_axis_name="s",
                                    num_cores=NC, num_subcores=NS),
        compiler_params=pltpu.CompilerParams(needs_layout_passes=False),
        scratch_types=[
            pltpu.VMEM((ROWS_PER_W * M,), jnp.int32),
            pltpu.VMEM((R * M, W), jnp.int32),
            pltpu.VMEM((R * M, W), jnp.int32),
            pltpu.VMEM((R, D), jnp.float32),
            pltpu.VMEM((R, D), jnp.float32),
            pltpu.VMEM((ROWS_PER_W,), jnp.float32),
            pltpu.SemaphoreType.DMA,
            pltpu.SemaphoreType.DMA,
            pltpu.SemaphoreType.DMA,
            pltpu.SemaphoreType.DMA,
        ],
    )


def _pack_halves(a):
    h = a.shape[1] // 2
    u = lax.bitcast_convert_type(a, jnp.uint32)
    r = u + jnp.uint32(0x7FFF) + ((u >> 16) & 1)
    lo = r[:, :h] >> 16
    hi = r[:, h:] & jnp.uint32(0xFFFF0000)
    return lax.bitcast_convert_type(lo | hi, jnp.int32)


def kernel(X, C, B):
    return _build()(X, _pack_halves(C), B.reshape(-1))


# speedup vs baseline: 1.1677x; 1.1677x over previous
import functools

import jax
import jax.numpy as jnp
from jax import lax
from jax.experimental import pallas as pl
from jax.experimental.pallas import tpu as pltpu, tpu_sc as plsc

N, D, M, V = 32768, 256, 8, 8192
NC, NS, L = 2, 16, 16
NW = NC * NS
ROWS_PER_W = N // NW
R = 16
BLOCKS = ROWS_PER_W // R
GIDX = 128
NG = R * M // GIDX
W = D // 2


def _sc_body(x_hbm, c_hbm, b_hbm, out_hbm,
             idx_all, rows0, rows1, x0, x1, out_v, gs0, gs1, xs0, xs1):
    wid = lax.axis_index("c") * NS + lax.axis_index("s")
    row0 = wid * ROWS_PER_W

    for j in range(M):
        pltpu.sync_copy(b_hbm.at[j, pl.ds(row0, ROWS_PER_W)],
                        idx_all.at[pl.ds(j * ROWS_PER_W, ROWS_PER_W)])

    rows = (rows0, rows1)
    xbuf = (x0, x1)
    gsem = (gs0, gs1)
    xsem = (xs0, xs1)

    def idx_slice(blk, j):
        off = pl.multiple_of(j * ROWS_PER_W + blk * R, R)
        return idx_all.at[pl.ds(off, R)]

    def start(blk, buf):
        base = row0 + blk * R
        for j in range(M):
            pltpu.async_copy(c_hbm.at[idx_slice(blk, j)],
                             rows[buf].at[pl.ds(j * R, R)], gsem[buf])
        pltpu.async_copy(x_hbm.at[pl.ds(base, R)], xbuf[buf], xsem[buf])

    def wait(blk, buf):
        for j in range(M):
            pltpu.make_async_copy(
                c_hbm.at[idx_slice(blk, j)],
                rows[buf].at[pl.ds(j * R, R)], gsem[buf]).wait()
        pltpu.make_async_copy(
            x_hbm.at[pl.ds(row0, R)], xbuf[buf], xsem[buf]).wait()

    lane = lax.iota(jnp.int32, L)

    def compute(blk, buf):
        rows_v = rows[buf]
        x_v = xbuf[buf]

        def _row(r, acc):
            err = jnp.zeros((L,), jnp.float32)
            for k in range(W // L):
                col = pl.ds(k * L, L)
                s = plsc.bitcast(rows_v[r, col], jnp.bfloat16)
                for j in range(1, M):
                    s = s + plsc.bitcast(rows_v[j * R + r, col], jnp.bfloat16)
                u = plsc.bitcast(s, jnp.int32)
                sa = plsc.bitcast(u << 16, jnp.float32)
                sb = plsc.bitcast(u & jnp.int32(-65536), jnp.float32)
                da = x_v[r, pl.ds(k * L, L)] - sa
                db = x_v[r, pl.ds(W + k * L, L)] - sb
                err = err + da * da
                err = err + db * db
            return jnp.where(lane == r, jnp.sum(err), acc)

        acc = lax.fori_loop(0, R, _row, jnp.zeros((L,), jnp.float32))
        out_v[pl.ds(blk * R, R)] = acc

    start(0, 0)
    start(1, 1)

    def _pair(p, carry):
        b0 = 2 * p
        wait(b0, 0)
        compute(b0, 0)

        @pl.when(b0 + 2 < BLOCKS)
        def _():
            start(b0 + 2, 0)

        wait(b0 + 1, 1)
        compute(b0 + 1, 1)

        @pl.when(b0 + 3 < BLOCKS)
        def _():
            start(b0 + 3, 1)

        return carry

    lax.fori_loop(0, BLOCKS // 2, _pair, 0)
    pltpu.sync_copy(out_v, out_hbm.at[pl.ds(row0, ROWS_PER_W)])


@functools.lru_cache(maxsize=1)
def _build():
    return pl.kernel(
        _sc_body,
        out_type=jax.ShapeDtypeStruct((N,), jnp.float32),
        mesh=plsc.VectorSubcoreMesh(core_axis_name="c", subcore_axis_name="s",
                                    num_cores=NC, num_subcores=NS),
        compiler_params=pltpu.CompilerParams(needs_layout_passes=False),
        scratch_types=[
            pltpu.VMEM((ROWS_PER_W * M,), jnp.int32),
            pltpu.VMEM((R * M, W), jnp.int32),
            pltpu.VMEM((R * M, W), jnp.int32),
            pltpu.VMEM((R, D), jnp.float32),
            pltpu.VMEM((R, D), jnp.float32),
            pltpu.VMEM((ROWS_PER_W,), jnp.float32),
            pltpu.SemaphoreType.DMA,
            pltpu.SemaphoreType.DMA,
            pltpu.SemaphoreType.DMA,
            pltpu.SemaphoreType.DMA,
        ],
    )


def _pack_halves(a):
    h = a.shape[1] // 2
    u = lax.bitcast_convert_type(a, jnp.uint32)
    r = u + jnp.uint32(0x7FFF) + ((u >> 16) & 1)
    lo = r[:, :h] >> 16
    hi = r[:, h:] & jnp.uint32(0xFFFF0000)
    return lax.bitcast_convert_type(lo | hi, jnp.int32)


def kernel(X, C, B):
    return _build()(X, _pack_halves(C), B.T)


# speedup vs baseline: 1.1710x; 1.0028x over previous
import functools

import jax
import jax.numpy as jnp
from jax import lax
from jax.experimental import pallas as pl
from jax.experimental.pallas import tpu as pltpu, tpu_sc as plsc

N, D, M, V = 32768, 256, 8, 8192
NC, NS, L = 2, 16, 16
NW = NC * NS
ROWS_PER_W = N // NW
R = 16
BLOCKS = ROWS_PER_W // R
GIDX = 128
NG = R * M // GIDX
W = D // 2


def _sc_body(x_hbm, c_hbm, b_hbm, out_hbm,
             idx_all, rows0, rows1, x0, x1, out_v, gs0, gs1, xs0, xs1):
    wid = lax.axis_index("c") * NS + lax.axis_index("s")
    row0 = wid * ROWS_PER_W

    for j in range(M):
        pltpu.sync_copy(b_hbm.at[j, pl.ds(row0, ROWS_PER_W)],
                        idx_all.at[pl.ds(j * ROWS_PER_W, ROWS_PER_W)])

    rows = (rows0, rows1)
    xbuf = (x0, x1)
    gsem = (gs0, gs1)
    xsem = (xs0, xs1)

    def idx_slice(blk, j):
        off = pl.multiple_of(j * ROWS_PER_W + blk * R, R)
        return idx_all.at[pl.ds(off, R)]

    def start(blk, buf):
        base = row0 + blk * R
        for j in range(M):
            pltpu.async_copy(c_hbm.at[idx_slice(blk, j)],
                             rows[buf].at[pl.ds(j * R, R)], gsem[buf])
        pltpu.async_copy(x_hbm.at[pl.ds(base, R)], xbuf[buf], xsem[buf])

    def wait(blk, buf):
        for j in range(M):
            pltpu.make_async_copy(
                c_hbm.at[idx_slice(blk, j)],
                rows[buf].at[pl.ds(j * R, R)], gsem[buf]).wait()
        pltpu.make_async_copy(
            x_hbm.at[pl.ds(row0, R)], xbuf[buf], xsem[buf]).wait()

    lane = lax.iota(jnp.int32, L)

    def compute(blk, buf):
        rows_v = rows[buf]
        x_v = xbuf[buf]

        def _row(r, acc):
            err = jnp.zeros((L,), jnp.float32)
            for k in range(W // L):
                col = pl.ds(k * L, L)
                c = [plsc.bitcast(rows_v[j * R + r, col], jnp.bfloat16)
                     for j in range(M)]
                while len(c) > 1:
                    c = [a + b for a, b in zip(c[::2], c[1::2])]
                s = c[0]
                u = plsc.bitcast(s, jnp.int32)
                sa = plsc.bitcast(u << 16, jnp.float32)
                sb = plsc.bitcast(u & jnp.int32(-65536), jnp.float32)
                da = x_v[r, pl.ds(k * L, L)] - sa
                db = x_v[r, pl.ds(W + k * L, L)] - sb
                err = err + da * da
                err = err + db * db
            return jnp.where(lane == r, jnp.sum(err), acc)

        acc = lax.fori_loop(0, R, _row, jnp.zeros((L,), jnp.float32))
        out_v[pl.ds(blk * R, R)] = acc

    start(0, 0)
    start(1, 1)

    def _pair(p, carry):
        b0 = 2 * p
        wait(b0, 0)
        compute(b0, 0)

        @pl.when(b0 + 2 < BLOCKS)
        def _():
            start(b0 + 2, 0)

        wait(b0 + 1, 1)
        compute(b0 + 1, 1)

        @pl.when(b0 + 3 < BLOCKS)
        def _():
            start(b0 + 3, 1)

        return carry

    lax.fori_loop(0, BLOCKS // 2, _pair, 0)
    pltpu.sync_copy(out_v, out_hbm.at[pl.ds(row0, ROWS_PER_W)])


@functools.lru_cache(maxsize=1)
def _build():
    return pl.kernel(
        _sc_body,
        out_type=jax.ShapeDtypeStruct((N,), jnp.float32),
        mesh=plsc.VectorSubcoreMesh(core_axis_name="c", subcore_axis_name="s",
                                    num_cores=NC, num_subcores=NS),
        compiler_params=pltpu.CompilerParams(needs_layout_passes=False),
        scratch_types=[
            pltpu.VMEM((ROWS_PER_W * M,), jnp.int32),
            pltpu.VMEM((R * M, W), jnp.int32),
            pltpu.VMEM((R * M, W), jnp.int32),
            pltpu.VMEM((R, D), jnp.float32),
            pltpu.VMEM((R, D), jnp.float32),
            pltpu.VMEM((ROWS_PER_W,), jnp.float32),
            pltpu.SemaphoreType.DMA,
            pltpu.SemaphoreType.DMA,
            pltpu.SemaphoreType.DMA,
            pltpu.SemaphoreType.DMA,
        ],
    )


def _pack_halves(a):
    h = a.shape[1] // 2
    u = lax.bitcast_convert_type(a, jnp.uint32)
    r = u + jnp.uint32(0x7FFF) + ((u >> 16) & 1)
    lo = r[:, :h] >> 16
    hi = r[:, h:] & jnp.uint32(0xFFFF0000)
    return lax.bitcast_convert_type(lo | hi, jnp.int32)


def kernel(X, C, B):
    return _build()(X, _pack_halves(C), B.T)


# speedup vs baseline: 1.2157x; 1.0383x over previous
import functools

import jax
import jax.numpy as jnp
from jax import lax
from jax.experimental import pallas as pl
from jax.experimental.pallas import tpu as pltpu, tpu_sc as plsc

N, D, M, V = 32768, 256, 8, 8192
NC, NS, L = 2, 16, 16
NW = NC * NS
ROWS_PER_W = N // NW
R = 16
BLOCKS = ROWS_PER_W // R
GIDX = 128
NG = R * M // GIDX
W = D // 2


def _sc_body(x_hbm, c_hbm, b_hbm, out_hbm,
             idx_all, rows0, rows1, x0, x1, out_v, gs0, gs1, xs0, xs1):
    wid = lax.axis_index("c") * NS + lax.axis_index("s")
    row0 = wid * ROWS_PER_W

    copies = [
        pltpu.async_copy(b_hbm.at[j, pl.ds(row0, ROWS_PER_W)],
                         idx_all.at[pl.ds(j * ROWS_PER_W, ROWS_PER_W)], xs0)
        for j in range(M)
    ]
    for c in copies:
        c.wait()

    rows = (rows0, rows1)
    xbuf = (x0, x1)
    gsem = (gs0, gs1)
    xsem = (xs0, xs1)

    def idx_slice(blk, j):
        off = pl.multiple_of(j * ROWS_PER_W + blk * R, R)
        return idx_all.at[pl.ds(off, R)]

    def start(blk, buf):
        base = row0 + blk * R
        for j in range(M):
            pltpu.async_copy(c_hbm.at[idx_slice(blk, j)],
                             rows[buf].at[pl.ds(j * R, R)], gsem[buf])
        pltpu.async_copy(x_hbm.at[pl.ds(base, R)], xbuf[buf], xsem[buf])

    def wait(blk, buf):
        for j in range(M):
            pltpu.make_async_copy(
                c_hbm.at[idx_slice(blk, j)],
                rows[buf].at[pl.ds(j * R, R)], gsem[buf]).wait()
        pltpu.make_async_copy(
            x_hbm.at[pl.ds(row0, R)], xbuf[buf], xsem[buf]).wait()

    lane = lax.iota(jnp.int32, L)

    def compute(blk, buf):
        rows_v = rows[buf]
        x_v = xbuf[buf]

        def _row(r, acc):
            err = jnp.zeros((L,), jnp.float32)
            for k in range(W // L):
                col = pl.ds(k * L, L)
                c = [plsc.bitcast(rows_v[j * R + r, col], jnp.bfloat16)
                     for j in range(M)]
                while len(c) > 1:
                    c = [a + b for a, b in zip(c[::2], c[1::2])]
                s = c[0]
                u = plsc.bitcast(s, jnp.int32)
                sa = plsc.bitcast(u << 16, jnp.float32)
                sb = plsc.bitcast(u & jnp.int32(-65536), jnp.float32)
                da = x_v[r, pl.ds(k * L, L)] - sa
                db = x_v[r, pl.ds(W + k * L, L)] - sb
                err = err + da * da
                err = err + db * db
            return jnp.where(lane == r, jnp.sum(err), acc)

        acc = lax.fori_loop(0, R, _row, jnp.zeros((L,), jnp.float32))
        out_v[pl.ds(blk * R, R)] = acc

    start(0, 0)
    start(1, 1)

    def _pair(p, carry):
        b0 = 2 * p
        wait(b0, 0)
        compute(b0, 0)

        @pl.when(b0 + 2 < BLOCKS)
        def _():
            start(b0 + 2, 0)

        wait(b0 + 1, 1)
        compute(b0 + 1, 1)

        @pl.when(b0 + 3 < BLOCKS)
        def _():
            start(b0 + 3, 1)

        return carry

    lax.fori_loop(0, BLOCKS // 2, _pair, 0)
    pltpu.sync_copy(out_v, out_hbm.at[pl.ds(row0, ROWS_PER_W)])


@functools.lru_cache(maxsize=1)
def _build():
    return pl.kernel(
        _sc_body,
        out_type=jax.ShapeDtypeStruct((N,), jnp.float32),
        mesh=plsc.VectorSubcoreMesh(core_axis_name="c", subcore_axis_name="s",
                                    num_cores=NC, num_subcores=NS),
        compiler_params=pltpu.CompilerParams(needs_layout_passes=False),
        scratch_types=[
            pltpu.VMEM((ROWS_PER_W * M,), jnp.int32),
            pltpu.VMEM((R * M, W), jnp.int32),
            pltpu.VMEM((R * M, W), jnp.int32),
            pltpu.VMEM((R, D), jnp.float32),
            pltpu.VMEM((R, D), jnp.float32),
            pltpu.VMEM((ROWS_PER_W,), jnp.float32),
            pltpu.SemaphoreType.DMA,
            pltpu.SemaphoreType.DMA,
            pltpu.SemaphoreType.DMA,
            pltpu.SemaphoreType.DMA,
        ],
    )


def _pack_halves(a):
    h = a.shape[1] // 2
    u = lax.bitcast_convert_type(a, jnp.uint32)

    def rne(x):
        return x + jnp.uint32(0x7FFF) + ((x >> 16) & 1)

    lo = rne(u[:, :h]) >> 16
    hi = rne(u[:, h:]) & jnp.uint32(0xFFFF0000)
    return lax.bitcast_convert_type(lo | hi, jnp.int32)


def kernel(X, C, B):
    return _build()(X, _pack_halves(C), B.T)


# speedup vs baseline: 1.2304x; 1.0121x over previous
import functools

import jax
import jax.numpy as jnp
from jax import lax
from jax.experimental import pallas as pl
from jax.experimental.pallas import tpu as pltpu, tpu_sc as plsc

N, D, M, V = 32768, 256, 8, 8192
NC, NS, L = 2, 16, 16
NW = NC * NS
ROWS_PER_W = N // NW
R = 16
BLOCKS = ROWS_PER_W // R
GIDX = 128
NG = R * M // GIDX
W = D // 2


def _sc_body(x_hbm, c_hbm, b_hbm, out_hbm,
             idx_all, rows0, rows1, x0, x1, out_v, gs0, gs1, xs0, xs1):
    wid = lax.axis_index("c") * NS + lax.axis_index("s")
    row0 = wid * ROWS_PER_W

    copies = [
        pltpu.async_copy(b_hbm.at[j, pl.ds(row0, ROWS_PER_W)],
                         idx_all.at[pl.ds(j * ROWS_PER_W, ROWS_PER_W)], xs0)
        for j in range(M)
    ]
    for c in copies:
        c.wait()

    rows = (rows0, rows1)
    xbuf = (x0, x1)
    gsem = (gs0, gs1)
    xsem = (xs0, xs1)

    def idx_slice(blk, j):
        off = pl.multiple_of(j * ROWS_PER_W + blk * R, R)
        return idx_all.at[pl.ds(off, R)]

    def start(blk, buf):
        base = row0 + blk * R
        for j in range(M):
            pltpu.async_copy(c_hbm.at[idx_slice(blk, j)],
                             rows[buf].at[pl.ds(j * R, R)], gsem[buf])
        pltpu.async_copy(x_hbm.at[pl.ds(base, R)], xbuf[buf], xsem[buf])

    def wait(blk, buf):
        for j in range(M):
            pltpu.make_async_copy(
                c_hbm.at[idx_slice(blk, j)],
                rows[buf].at[pl.ds(j * R, R)], gsem[buf]).wait()
        pltpu.make_async_copy(
            x_hbm.at[pl.ds(row0, R)], xbuf[buf], xsem[buf]).wait()

    lane = lax.iota(jnp.int32, L)

    def compute(blk, buf):
        rows_v = rows[buf]
        x_v = xbuf[buf]

        def _row(r, acc):
            err = jnp.zeros((L,), jnp.float32)
            for k in range(W // L):
                col = pl.ds(k * L, L)
                c = [plsc.bitcast(rows_v[j * R + r, col], jnp.bfloat16)
                     for j in range(M)]
                while len(c) > 1:
                    c = [a + b for a, b in zip(c[::2], c[1::2])]
                s = c[0]
                u = plsc.bitcast(s, jnp.int32)
                sa = plsc.bitcast(u << 16, jnp.float32)
                sb = plsc.bitcast(u & jnp.int32(-65536), jnp.float32)
                da = x_v[r, pl.ds(k * L, L)] - sa
                db = x_v[r, pl.ds(W + k * L, L)] - sb
                err = err + da * da
                err = err + db * db
            return jnp.where(lane == r, jnp.sum(err), acc)

        acc = lax.fori_loop(0, R, _row, jnp.zeros((L,), jnp.float32))
        out_v[pl.ds(blk * R, R)] = acc

    start(0, 0)
    start(1, 1)

    def _pair(p, carry):
        b0 = 2 * p
        wait(b0, 0)
        compute(b0, 0)

        @pl.when(b0 + 2 < BLOCKS)
        def _():
            start(b0 + 2, 0)

        wait(b0 + 1, 1)
        compute(b0 + 1, 1)

        @pl.when(b0 + 3 < BLOCKS)
        def _():
            start(b0 + 3, 1)

        return carry

    lax.fori_loop(0, BLOCKS // 2, _pair, 0)
    pltpu.sync_copy(out_v, out_hbm.at[pl.ds(row0, ROWS_PER_W)])


@functools.lru_cache(maxsize=1)
def _build():
    return pl.kernel(
        _sc_body,
        out_type=jax.ShapeDtypeStruct((N,), jnp.float32),
        mesh=plsc.VectorSubcoreMesh(core_axis_name="c", subcore_axis_name="s",
                                    num_cores=NC, num_subcores=NS),
        compiler_params=pltpu.CompilerParams(needs_layout_passes=False),
        scratch_types=[
            pltpu.VMEM((ROWS_PER_W * M,), jnp.int32),
            pltpu.VMEM((R * M, W), jnp.int32),
            pltpu.VMEM((R * M, W), jnp.int32),
            pltpu.VMEM((R, D), jnp.float32),
            pltpu.VMEM((R, D), jnp.float32),
            pltpu.VMEM((ROWS_PER_W,), jnp.float32),
            pltpu.SemaphoreType.DMA,
            pltpu.SemaphoreType.DMA,
            pltpu.SemaphoreType.DMA,
            pltpu.SemaphoreType.DMA,
        ],
    )


def _pack_halves(a):
    h = a.shape[1] // 2

    def rne(x):
        u = lax.bitcast_convert_type(x, jnp.uint32)
        return u + jnp.uint32(0x7FFF) + ((u >> 16) & 1)

    lo = rne(a[:, :h]) >> 16
    hi = rne(a[:, h:]) & jnp.uint32(0xFFFF0000)
    return lax.bitcast_convert_type(lo | hi, jnp.int32)


def kernel(X, C, B):
    return _build()(X, _pack_halves(C), B.T)
